# baseline (speedup 1.0000x reference)
import functools

import jax
import jax.numpy as jnp
from jax import lax
from jax.experimental import pallas as pl
from jax.experimental.pallas import tpu as pltpu
from jax.experimental.pallas import tpu_sc as plsc

B = 4096
EMB = 64
H = 2 * EMB

_NC = 2
_NS = 16
_NW = _NC * _NS
_BPW = B // _NW
_NL = 16


def _sc_gather(track_table, artist_table, W1, track_id, artist_id, genres):
  mesh = plsc.VectorSubcoreMesh(core_axis_name="c", subcore_axis_name="s")

  @functools.partial(
      pl.kernel,
      mesh=mesh,
      out_type=(
          jax.ShapeDtypeStruct((B, H), jnp.bfloat16),
          jax.ShapeDtypeStruct((B, H), jnp.bfloat16),
          jax.ShapeDtypeStruct((B, H), jnp.float32),
      ),
      scratch_types=[
          pltpu.VMEM((_BPW,), jnp.int32),
          pltpu.VMEM((_BPW,), jnp.int32),
          pltpu.VMEM((_BPW,), jnp.int32),
          pltpu.VMEM((_BPW, H), jnp.bfloat16),
          pltpu.VMEM((_BPW, H), jnp.bfloat16),
          pltpu.VMEM((_BPW, H), jnp.float32),
          pltpu.SemaphoreType.DMA,
          pltpu.SemaphoreType.DMA,
          pltpu.SemaphoreType.DMA,
      ],
      compiler_params=pltpu.CompilerParams(use_tc_tiling_on_sc=False),
  )
  def k(tt, at, w1, tid, aid, gid, t_out, a_out, g_out,
        tix, aix, gix, trows, arows, grows, sem_t, sem_a, sem_g):
    wid = lax.axis_index("s") * _NC + lax.axis_index("c")
    base = wid * _BPW
    pltpu.sync_copy(tid.at[pl.ds(base, _BPW)], tix)
    pltpu.sync_copy(aid.at[pl.ds(base, _BPW)], aix)
    pltpu.sync_copy(gid.at[pl.ds(base, _BPW)], gix)
    for j in range(_BPW // _NL):
      sl = pl.ds(j * _NL, _NL)
      gix[sl] = gix[sl] + 2 * EMB
    ct = pltpu.async_copy(tt.at[tix], trows, sem_t)
    ca = pltpu.async_copy(at.at[aix], arows, sem_a)
    cg = pltpu.async_copy(w1.at[gix], grows, sem_g)
    ct.wait()
    pltpu.sync_copy(trows, t_out.at[pl.ds(base, _BPW)])
    ca.wait()
    pltpu.sync_copy(arows, a_out.at[pl.ds(base, _BPW)])
    cg.wait()
    pltpu.sync_copy(grows, g_out.at[pl.ds(base, _BPW)])

  return k(track_table, artist_table, W1, track_id, artist_id, genres)


V = 100001
_CB = 2048
_NB = (V + _CB - 1) // _CB


def _detile_body(t_ref, a_ref, to_ref, ao_ref):
  t = t_ref[...].T.astype(jnp.bfloat16)
  a = a_ref[...].T.astype(jnp.bfloat16)
  to_ref[...] = jnp.concatenate([t, t], axis=1).reshape(H * _CB)
  ao_ref[...] = jnp.concatenate([a, a], axis=1).reshape(H * _CB)


def _tc_detile(tT, aT):
  return pl.pallas_call(
      _detile_body,
      grid=(_NB,),
      in_specs=[
          pl.BlockSpec((EMB, _CB), lambda i: (0, i)),
          pl.BlockSpec((EMB, _CB), lambda i: (0, i)),
      ],
      out_specs=[
          pl.BlockSpec((H * _CB,), lambda i: (i,)),
          pl.BlockSpec((H * _CB,), lambda i: (i,)),
      ],
      out_shape=[
          jax.ShapeDtypeStruct((V * H,), jnp.bfloat16),
          jax.ShapeDtypeStruct((V * H,), jnp.bfloat16),
      ],
  )(tT, aT)


_BLK = 512


def _tc_body(t_ref, a_ref, g_ref, au_ref, mean_ref, var_ref,
             w1_ref, b1_ref, w2_ref, b2_ref, o_ref):
  audio = (au_ref[...] - mean_ref[...]) * lax.rsqrt(var_ref[...])
  h = g_ref[...] + b1_ref[...]
  h += jnp.dot(t_ref[:, :EMB].astype(jnp.float32), w1_ref[:EMB, :],
               preferred_element_type=jnp.float32)
  h += jnp.dot(a_ref[:, :EMB].astype(jnp.float32), w1_ref[EMB:2 * EMB, :],
               preferred_element_type=jnp.float32)
  h += jnp.dot(audio, w1_ref[2 * EMB + 101:, :],
               preferred_element_type=jnp.float32)
  h = jnp.maximum(h, 0.0)
  o_ref[...] = jnp.dot(h, w2_ref[...], preferred_element_type=jnp.float32) + b2_ref[...]


def _tc_dense(t2, a2, grows, audio, norm_mean, norm_var, W1, b1, W2, b2):
  n_blk = B // _BLK
  full = lambda shape: pl.BlockSpec(shape, lambda i: (0, 0))
  return pl.pallas_call(
      _tc_body,
      grid=(n_blk,),
      in_specs=[
          pl.BlockSpec((_BLK, H), lambda i: (i, 0)),
          pl.BlockSpec((_BLK, H), lambda i: (i, 0)),
          pl.BlockSpec((_BLK, H), lambda i: (i, 0)),
          pl.BlockSpec((_BLK, 8), lambda i: (i, 0)),
          full((1, 8)),
          full((1, 8)),
          full((237, H)),
          full((1, H)),
          full((H, EMB)),
          full((1, EMB)),
      ],
      out_specs=pl.BlockSpec((_BLK, EMB), lambda i: (i, 0)),
      out_shape=jax.ShapeDtypeStruct((B, EMB), jnp.float32),
  )(t2, a2, grows, audio, norm_mean, norm_var, W1, b1, W2, b2)


def kernel(track_id, artist_id, genres, danceability, energy, instrumentalness,
           acousticness, valence, speechiness, loudness, liveness,
           norm_mean, norm_var, track_table, artist_table, W1, b1, W2, b2):
  tid = track_id.astype(jnp.int32)
  aid = artist_id.astype(jnp.int32)
  gid = genres.astype(jnp.int32)
  t_flat, a_flat = _tc_detile(track_table.T, artist_table.T)
  t2, a2, grows = _sc_gather(t_flat.reshape(V, H), a_flat.reshape(V, H),
                             W1, tid, aid, gid)
  audio = jnp.stack([danceability, energy, instrumentalness, acousticness,
                     valence, speechiness, loudness, liveness], axis=1)
  return _tc_dense(t2, a2, grows, audio,
                   norm_mean.reshape(1, 8), norm_var.reshape(1, 8),
                   W1, b1.reshape(1, H), W2, b2.reshape(1, EMB))


# speedup vs baseline: 2.7775x; 2.7775x over previous
import functools

import jax
import jax.numpy as jnp
from jax import lax
from jax.experimental import pallas as pl
from jax.experimental.pallas import tpu as pltpu
from jax.experimental.pallas import tpu_sc as plsc

B = 4096
EMB = 64
H = 2 * EMB

_NC = 2
_NS = 16
_NW = _NC * _NS
_BPW = B // _NW
_NL = 16


def _sc_gather(track_table, artist_table, W1, track_id, artist_id, genres):
  mesh = plsc.VectorSubcoreMesh(core_axis_name="c", subcore_axis_name="s")

  @functools.partial(
      pl.kernel,
      mesh=mesh,
      out_type=(
          jax.ShapeDtypeStruct((B, H), jnp.float32),
          jax.ShapeDtypeStruct((B, H), jnp.float32),
          jax.ShapeDtypeStruct((B, H), jnp.float32),
      ),
      scratch_types=[
          pltpu.VMEM((_BPW,), jnp.int32),
          pltpu.VMEM((_BPW,), jnp.int32),
          pltpu.VMEM((_BPW,), jnp.int32),
          pltpu.VMEM((_BPW, H), jnp.float32),
          pltpu.VMEM((_BPW, H), jnp.float32),
          pltpu.VMEM((_BPW, H), jnp.float32),
          pltpu.SemaphoreType.DMA,
          pltpu.SemaphoreType.DMA,
          pltpu.SemaphoreType.DMA,
      ],
      compiler_params=pltpu.CompilerParams(use_tc_tiling_on_sc=False),
  )
  def k(tt, at, w1, tid, aid, gid, t_out, a_out, g_out,
        tix, aix, gix, trows, arows, grows, sem_t, sem_a, sem_g):
    wid = lax.axis_index("s") * _NC + lax.axis_index("c")
    base = wid * _BPW
    pltpu.sync_copy(tid.at[pl.ds(base, _BPW)], tix)
    pltpu.sync_copy(aid.at[pl.ds(base, _BPW)], aix)
    pltpu.sync_copy(gid.at[pl.ds(base, _BPW)], gix)
    for j in range(_BPW // _NL):
      sl = pl.ds(j * _NL, _NL)
      gix[sl] = gix[sl] + 2 * EMB
    ct = pltpu.async_copy(tt.at[tix], trows, sem_t)
    ca = pltpu.async_copy(at.at[aix], arows, sem_a)
    cg = pltpu.async_copy(w1.at[gix], grows, sem_g)
    ct.wait()
    pltpu.sync_copy(trows, t_out.at[pl.ds(base, _BPW)])
    ca.wait()
    pltpu.sync_copy(arows, a_out.at[pl.ds(base, _BPW)])
    cg.wait()
    pltpu.sync_copy(grows, g_out.at[pl.ds(base, _BPW)])

  return k(track_table, artist_table, W1, track_id, artist_id, genres)


V = 100001
_CB = 2048
_NB = (V + _CB - 1) // _CB


def _detile_body(t_ref, a_ref, to_ref, ao_ref):
  t = t_ref[...].T
  a = a_ref[...].T
  to_ref[...] = jnp.concatenate([t, t], axis=1).reshape(H * _CB)
  ao_ref[...] = jnp.concatenate([a, a], axis=1).reshape(H * _CB)


def _tc_detile(tT, aT):
  return pl.pallas_call(
      _detile_body,
      grid=(_NB,),
      in_specs=[
          pl.BlockSpec((EMB, _CB), lambda i: (0, i)),
          pl.BlockSpec((EMB, _CB), lambda i: (0, i)),
      ],
      out_specs=[
          pl.BlockSpec((H * _CB,), lambda i: (i,)),
          pl.BlockSpec((H * _CB,), lambda i: (i,)),
      ],
      out_shape=[
          jax.ShapeDtypeStruct((V * H,), jnp.float32),
          jax.ShapeDtypeStruct((V * H,), jnp.float32),
      ],
  )(tT, aT)


_BLK = 512


def _tc_body(t_ref, a_ref, g_ref, au_ref, mean_ref, var_ref,
             w1_ref, b1_ref, w2_ref, b2_ref, o_ref):
  audio = (au_ref[...] - mean_ref[...]) * lax.rsqrt(var_ref[...])
  h = g_ref[...] + b1_ref[...]
  h += jnp.dot(t_ref[:, :EMB], w1_ref[:EMB, :],
               preferred_element_type=jnp.float32)
  h += jnp.dot(a_ref[:, :EMB], w1_ref[EMB:2 * EMB, :],
               preferred_element_type=jnp.float32)
  h += jnp.dot(audio, w1_ref[2 * EMB + 101:, :],
               preferred_element_type=jnp.float32)
  h = jnp.maximum(h, 0.0)
  o_ref[...] = jnp.dot(h, w2_ref[...], preferred_element_type=jnp.float32) + b2_ref[...]


def _tc_dense(t2, a2, grows, audio, norm_mean, norm_var, W1, b1, W2, b2):
  n_blk = B // _BLK
  full = lambda shape: pl.BlockSpec(shape, lambda i: (0, 0))
  return pl.pallas_call(
      _tc_body,
      grid=(n_blk,),
      in_specs=[
          pl.BlockSpec((_BLK, H), lambda i: (i, 0)),
          pl.BlockSpec((_BLK, H), lambda i: (i, 0)),
          pl.BlockSpec((_BLK, H), lambda i: (i, 0)),
          pl.BlockSpec((_BLK, 8), lambda i: (i, 0)),
          full((1, 8)),
          full((1, 8)),
          full((237, H)),
          full((1, H)),
          full((H, EMB)),
          full((1, EMB)),
      ],
      out_specs=pl.BlockSpec((_BLK, EMB), lambda i: (i, 0)),
      out_shape=jax.ShapeDtypeStruct((B, EMB), jnp.float32),
  )(t2, a2, grows, audio, norm_mean, norm_var, W1, b1, W2, b2)


def kernel(track_id, artist_id, genres, danceability, energy, instrumentalness,
           acousticness, valence, speechiness, loudness, liveness,
           norm_mean, norm_var, track_table, artist_table, W1, b1, W2, b2):
  tid = track_id.astype(jnp.int32)
  aid = artist_id.astype(jnp.int32)
  gid = genres.astype(jnp.int32)
  t_flat, a_flat = _tc_detile(track_table.T, artist_table.T)
  t2, a2, grows = _sc_gather(t_flat.reshape(V, H), a_flat.reshape(V, H),
                             W1, tid, aid, gid)
  audio = jnp.stack([danceability, energy, instrumentalness, acousticness,
                     valence, speechiness, loudness, liveness], axis=1)
  return _tc_dense(t2, a2, grows, audio,
                   norm_mean.reshape(1, 8), norm_var.reshape(1, 8),
                   W1, b1.reshape(1, H), W2, b2.reshape(1, EMB))


# speedup vs baseline: 3.0550x; 1.0999x over previous
import functools

import jax
import jax.numpy as jnp
from jax import lax
from jax.experimental import pallas as pl
from jax.experimental.pallas import tpu as pltpu
from jax.experimental.pallas import tpu_sc as plsc

B = 4096
EMB = 64
H = 2 * EMB

_NC = 2
_NS = 16
_NW = _NC * _NS
_BPW = B // _NW
_NL = 16


def _sc_gather(track_table, artist_table, W1, track_id, artist_id, genres):
  mesh = plsc.VectorSubcoreMesh(core_axis_name="c", subcore_axis_name="s")

  @functools.partial(
      pl.kernel,
      mesh=mesh,
      out_type=(
          jax.ShapeDtypeStruct((B, H), jnp.float32),
          jax.ShapeDtypeStruct((B, H), jnp.float32),
          jax.ShapeDtypeStruct((B, H), jnp.float32),
      ),
      scratch_types=[
          pltpu.VMEM((_BPW,), jnp.int32),
          pltpu.VMEM((_BPW,), jnp.int32),
          pltpu.VMEM((_BPW,), jnp.int32),
          pltpu.VMEM((_BPW, H), jnp.float32),
          pltpu.VMEM((_BPW, H), jnp.float32),
          pltpu.VMEM((_BPW, H), jnp.float32),
          pltpu.SemaphoreType.DMA,
          pltpu.SemaphoreType.DMA,
          pltpu.SemaphoreType.DMA,
      ],
      compiler_params=pltpu.CompilerParams(use_tc_tiling_on_sc=False),
  )
  def k(tt, at, w1, tid, aid, gid, t_out, a_out, g_out,
        tix, aix, gix, trows, arows, grows, sem_t, sem_a, sem_g):
    wid = lax.axis_index("s") * _NC + lax.axis_index("c")
    base = wid * _BPW
    pltpu.sync_copy(tid.at[pl.ds(base, _BPW)], tix)
    pltpu.sync_copy(aid.at[pl.ds(base, _BPW)], aix)
    pltpu.sync_copy(gid.at[pl.ds(base, _BPW)], gix)
    for j in range(_BPW // _NL):
      sl = pl.ds(j * _NL, _NL)
      gix[sl] = gix[sl] + 2 * EMB
    ct = pltpu.async_copy(tt.at[tix], trows, sem_t)
    ca = pltpu.async_copy(at.at[aix], arows, sem_a)
    cg = pltpu.async_copy(w1.at[gix], grows, sem_g)
    ct.wait()
    pltpu.sync_copy(trows, t_out.at[pl.ds(base, _BPW)])
    ca.wait()
    pltpu.sync_copy(arows, a_out.at[pl.ds(base, _BPW)])
    cg.wait()
    pltpu.sync_copy(grows, g_out.at[pl.ds(base, _BPW)])

  return k(track_table, artist_table, W1, track_id, artist_id, genres)


V = 100001
_CB = 4096
_NB = (V + _CB - 1) // _CB


def _detile_body(t_ref, a_ref, to_ref, ao_ref):
  t = t_ref[...].T
  a = a_ref[...].T
  to_ref[...] = jnp.concatenate([t, t], axis=1).reshape(H * _CB)
  ao_ref[...] = jnp.concatenate([a, a], axis=1).reshape(H * _CB)


def _tc_detile(tT, aT):
  return pl.pallas_call(
      _detile_body,
      grid=(_NB,),
      in_specs=[
          pl.BlockSpec((EMB, _CB), lambda i: (0, i)),
          pl.BlockSpec((EMB, _CB), lambda i: (0, i)),
      ],
      out_specs=[
          pl.BlockSpec((H * _CB,), lambda i: (i,)),
          pl.BlockSpec((H * _CB,), lambda i: (i,)),
      ],
      out_shape=[
          jax.ShapeDtypeStruct((V * H,), jnp.float32),
          jax.ShapeDtypeStruct((V * H,), jnp.float32),
      ],
  )(tT, aT)


_BLK = 512


def _tc_body(t_ref, a_ref, g_ref, au_ref, mean_ref, var_ref,
             w1_ref, b1_ref, w2_ref, b2_ref, o_ref):
  audio = (au_ref[...] - mean_ref[...]) * lax.rsqrt(var_ref[...])
  h = g_ref[...] + b1_ref[...]
  h += jnp.dot(t_ref[:, :EMB], w1_ref[:EMB, :],
               preferred_element_type=jnp.float32)
  h += jnp.dot(a_ref[:, :EMB], w1_ref[EMB:2 * EMB, :],
               preferred_element_type=jnp.float32)
  h += jnp.dot(audio, w1_ref[2 * EMB + 101:, :],
               preferred_element_type=jnp.float32)
  h = jnp.maximum(h, 0.0)
  o_ref[...] = jnp.dot(h, w2_ref[...], preferred_element_type=jnp.float32) + b2_ref[...]


def _tc_dense(t2, a2, grows, audio, norm_mean, norm_var, W1, b1, W2, b2):
  n_blk = B // _BLK
  full = lambda shape: pl.BlockSpec(shape, lambda i: (0, 0))
  return pl.pallas_call(
      _tc_body,
      grid=(n_blk,),
      in_specs=[
          pl.BlockSpec((_BLK, H), lambda i: (i, 0)),
          pl.BlockSpec((_BLK, H), lambda i: (i, 0)),
          pl.BlockSpec((_BLK, H), lambda i: (i, 0)),
          pl.BlockSpec((_BLK, 8), lambda i: (i, 0)),
          full((1, 8)),
          full((1, 8)),
          full((237, H)),
          full((1, H)),
          full((H, EMB)),
          full((1, EMB)),
      ],
      out_specs=pl.BlockSpec((_BLK, EMB), lambda i: (i, 0)),
      out_shape=jax.ShapeDtypeStruct((B, EMB), jnp.float32),
  )(t2, a2, grows, audio, norm_mean, norm_var, W1, b1, W2, b2)


def kernel(track_id, artist_id, genres, danceability, energy, instrumentalness,
           acousticness, valence, speechiness, loudness, liveness,
           norm_mean, norm_var, track_table, artist_table, W1, b1, W2, b2):
  tid = track_id.astype(jnp.int32)
  aid = artist_id.astype(jnp.int32)
  gid = genres.astype(jnp.int32)
  t_flat, a_flat = _tc_detile(track_table.T, artist_table.T)
  t2, a2, grows = _sc_gather(t_flat.reshape(V, H), a_flat.reshape(V, H),
                             W1, tid, aid, gid)
  audio = jnp.stack([danceability, energy, instrumentalness, acousticness,
                     valence, speechiness, loudness, liveness], axis=1)
  return _tc_dense(t2, a2, grows, audio,
                   norm_mean.reshape(1, 8), norm_var.reshape(1, 8),
                   W1, b1.reshape(1, H), W2, b2.reshape(1, EMB))


# speedup vs baseline: 3.2543x; 1.0652x over previous
import functools

import jax
import jax.numpy as jnp
from jax import lax
from jax.experimental import pallas as pl
from jax.experimental.pallas import tpu as pltpu
from jax.experimental.pallas import tpu_sc as plsc

B = 4096
EMB = 64
H = 2 * EMB

_NC = 2
_NS = 16
_NW = _NC * _NS
_BPW = B // _NW
_NL = 16


def _sc_gather(track_table, artist_table, W1, track_id, artist_id, genres):
  mesh = plsc.VectorSubcoreMesh(core_axis_name="c", subcore_axis_name="s")

  @functools.partial(
      pl.kernel,
      mesh=mesh,
      out_type=(
          jax.ShapeDtypeStruct((B, H), jnp.float32),
          jax.ShapeDtypeStruct((B, H), jnp.float32),
          jax.ShapeDtypeStruct((B, H), jnp.float32),
      ),
      scratch_types=[
          pltpu.VMEM((_BPW,), jnp.int32),
          pltpu.VMEM((_BPW,), jnp.int32),
          pltpu.VMEM((_BPW,), jnp.int32),
          pltpu.VMEM((_BPW, H), jnp.float32),
          pltpu.VMEM((_BPW, H), jnp.float32),
          pltpu.VMEM((_BPW, H), jnp.float32),
          pltpu.SemaphoreType.DMA,
          pltpu.SemaphoreType.DMA,
          pltpu.SemaphoreType.DMA,
      ],
      compiler_params=pltpu.CompilerParams(use_tc_tiling_on_sc=False),
  )
  def k(tt, at, w1, tid, aid, gid, t_out, a_out, g_out,
        tix, aix, gix, trows, arows, grows, sem_t, sem_a, sem_g):
    wid = lax.axis_index("s") * _NC + lax.axis_index("c")
    base = wid * _BPW
    pltpu.sync_copy(tid.at[pl.ds(base, _BPW)], tix)
    pltpu.sync_copy(aid.at[pl.ds(base, _BPW)], aix)
    pltpu.sync_copy(gid.at[pl.ds(base, _BPW)], gix)
    for j in range(_BPW // _NL):
      sl = pl.ds(j * _NL, _NL)
      gix[sl] = gix[sl] + 2 * EMB
    ct = pltpu.async_copy(tt.at[tix], trows, sem_t)
    ca = pltpu.async_copy(at.at[aix], arows, sem_a)
    cg = pltpu.async_copy(w1.at[gix], grows, sem_g)
    ct.wait()
    pltpu.sync_copy(trows, t_out.at[pl.ds(base, _BPW)])
    ca.wait()
    pltpu.sync_copy(arows, a_out.at[pl.ds(base, _BPW)])
    cg.wait()
    pltpu.sync_copy(grows, g_out.at[pl.ds(base, _BPW)])

  return k(track_table, artist_table, W1, track_id, artist_id, genres)


V = 100001
_CB = 8192
_NB = (V + _CB - 1) // _CB


def _detile_body(t_ref, a_ref, to_ref, ao_ref):
  t = t_ref[...].T
  a = a_ref[...].T
  to_ref[...] = jnp.concatenate([t, t], axis=1).reshape(H * _CB)
  ao_ref[...] = jnp.concatenate([a, a], axis=1).reshape(H * _CB)


def _tc_detile(tT, aT):
  return pl.pallas_call(
      _detile_body,
      grid=(_NB,),
      in_specs=[
          pl.BlockSpec((EMB, _CB), lambda i: (0, i)),
          pl.BlockSpec((EMB, _CB), lambda i: (0, i)),
      ],
      out_specs=[
          pl.BlockSpec((H * _CB,), lambda i: (i,)),
          pl.BlockSpec((H * _CB,), lambda i: (i,)),
      ],
      out_shape=[
          jax.ShapeDtypeStruct((V * H,), jnp.float32),
          jax.ShapeDtypeStruct((V * H,), jnp.float32),
      ],
  )(tT, aT)


_BLK = 512


def _tc_body(t_ref, a_ref, g_ref, au_ref, mean_ref, var_ref,
             w1_ref, b1_ref, w2_ref, b2_ref, o_ref):
  audio = (au_ref[...] - mean_ref[...]) * lax.rsqrt(var_ref[...])
  h = g_ref[...] + b1_ref[...]
  h += jnp.dot(t_ref[:, :EMB], w1_ref[:EMB, :],
               preferred_element_type=jnp.float32)
  h += jnp.dot(a_ref[:, :EMB], w1_ref[EMB:2 * EMB, :],
               preferred_element_type=jnp.float32)
  h += jnp.dot(audio, w1_ref[2 * EMB + 101:, :],
               preferred_element_type=jnp.float32)
  h = jnp.maximum(h, 0.0)
  o_ref[...] = jnp.dot(h, w2_ref[...], preferred_element_type=jnp.float32) + b2_ref[...]


def _tc_dense(t2, a2, grows, audio, norm_mean, norm_var, W1, b1, W2, b2):
  n_blk = B // _BLK
  full = lambda shape: pl.BlockSpec(shape, lambda i: (0, 0))
  return pl.pallas_call(
      _tc_body,
      grid=(n_blk,),
      in_specs=[
          pl.BlockSpec((_BLK, H), lambda i: (i, 0)),
          pl.BlockSpec((_BLK, H), lambda i: (i, 0)),
          pl.BlockSpec((_BLK, H), lambda i: (i, 0)),
          pl.BlockSpec((_BLK, 8), lambda i: (i, 0)),
          full((1, 8)),
          full((1, 8)),
          full((237, H)),
          full((1, H)),
          full((H, EMB)),
          full((1, EMB)),
      ],
      out_specs=pl.BlockSpec((_BLK, EMB), lambda i: (i, 0)),
      out_shape=jax.ShapeDtypeStruct((B, EMB), jnp.float32),
  )(t2, a2, grows, audio, norm_mean, norm_var, W1, b1, W2, b2)


def kernel(track_id, artist_id, genres, danceability, energy, instrumentalness,
           acousticness, valence, speechiness, loudness, liveness,
           norm_mean, norm_var, track_table, artist_table, W1, b1, W2, b2):
  tid = track_id.astype(jnp.int32)
  aid = artist_id.astype(jnp.int32)
  gid = genres.astype(jnp.int32)
  t_flat, a_flat = _tc_detile(track_table.T, artist_table.T)
  t2, a2, grows = _sc_gather(t_flat.reshape(V, H), a_flat.reshape(V, H),
                             W1, tid, aid, gid)
  audio = jnp.stack([danceability, energy, instrumentalness, acousticness,
                     valence, speechiness, loudness, liveness], axis=1)
  return _tc_dense(t2, a2, grows, audio,
                   norm_mean.reshape(1, 8), norm_var.reshape(1, 8),
                   W1, b1.reshape(1, H), W2, b2.reshape(1, EMB))
